# speedup vs baseline: 2.9763x; 2.9763x over previous
import functools

import jax
import jax.numpy as jnp
from jax import lax
from jax.experimental import pallas as pl
from jax.experimental.pallas import tpu as pltpu
from jax.experimental.pallas import tpu_sc as plsc

VOCAB = 100000
EMBED = 128
BATCH = 4096
HIST = 50

NTOT = BATCH * HIST
NW = 32
PER_W = NTOT // NW
CHUNK = 128
NCHUNK = PER_W // CHUNK


@functools.partial(
    pl.kernel,
    mesh=plsc.VectorSubcoreMesh(core_axis_name="c", subcore_axis_name="s"),
    out_type=jax.ShapeDtypeStruct((NTOT, EMBED), jnp.float32),
    scratch_types=[
        pltpu.VMEM((NCHUNK, CHUNK), jnp.int32),
        pltpu.VMEM((CHUNK, EMBED), jnp.float32),
        pltpu.SemaphoreType.DMA,
    ],
)
def _embed_gather(idx_hbm, table_hbm, out_hbm, idx_v, rows_v, sem):
    wid = lax.axis_index("s") * 2 + lax.axis_index("c")
    base = wid * PER_W
    pltpu.sync_copy(idx_hbm.at[wid], idx_v)

    def body(j, carry):
        pltpu.async_copy(table_hbm.at[idx_v.at[j]], rows_v, sem).wait()
        pltpu.sync_copy(rows_v, out_hbm.at[pl.ds(base + j * CHUNK, CHUNK)])
        return carry

    lax.fori_loop(0, NCHUNK, body, 0)


def kernel(input_seqs, table):
    idx = input_seqs.reshape(NW, NCHUNK, CHUNK).astype(jnp.int32)
    out = _embed_gather(idx, table)
    return out.reshape(BATCH, HIST, EMBED)


# speedup vs baseline: 3.3343x; 1.1203x over previous
import functools

import jax
import jax.numpy as jnp
from jax import lax
from jax.experimental import pallas as pl
from jax.experimental.pallas import tpu as pltpu
from jax.experimental.pallas import tpu_sc as plsc

VOCAB = 100000
EMBED = 128
BATCH = 4096
HIST = 50

NTOT = BATCH * HIST
NW = 32
PER_W = NTOT // NW
CHUNK = 128
NCHUNK = PER_W // CHUNK


@functools.partial(
    pl.kernel,
    mesh=plsc.VectorSubcoreMesh(core_axis_name="c", subcore_axis_name="s"),
    out_type=jax.ShapeDtypeStruct((NTOT, EMBED), jnp.float32),
    scratch_types=[
        pltpu.VMEM((NCHUNK, CHUNK), jnp.int32),
        pltpu.VMEM((CHUNK, EMBED), jnp.float32),
        pltpu.VMEM((CHUNK, EMBED), jnp.float32),
        pltpu.SemaphoreType.DMA,
        pltpu.SemaphoreType.DMA,
        pltpu.SemaphoreType.DMA,
        pltpu.SemaphoreType.DMA,
    ],
)
def _embed_gather(idx_hbm, table_hbm, out_hbm, idx_v, r0, r1,
                  g0, g1, w0, w1):
    wid = lax.axis_index("s") * 2 + lax.axis_index("c")
    base = wid * PER_W
    bufs = (r0, r1)
    gsems = (g0, g1)
    wsems = (w0, w1)
    pltpu.sync_copy(idx_hbm.at[wid], idx_v)

    pltpu.async_copy(table_hbm.at[idx_v.at[0]], r0, g0)
    pltpu.async_copy(table_hbm.at[idx_v.at[1]], r1, g1)

    def body(i, carry):
        j = i * 2
        for b in range(2):
            c = j + b
            buf, gs, ws = bufs[b], gsems[b], wsems[b]
            pltpu.make_async_copy(table_hbm.at[idx_v.at[c]], buf, gs).wait()
            dst = out_hbm.at[pl.ds(base + c * CHUNK, CHUNK)]
            pltpu.async_copy(buf, dst, ws)

            @pl.when(c + 2 < NCHUNK)
            def _():
                pltpu.make_async_copy(buf, dst, ws).wait()
                pltpu.async_copy(table_hbm.at[idx_v.at[c + 2]], buf, gs)

        return carry

    lax.fori_loop(0, NCHUNK // 2, body, 0)

    for b in range(2):
        c = NCHUNK - 2 + b
        dst = out_hbm.at[pl.ds(base + c * CHUNK, CHUNK)]
        pltpu.make_async_copy(bufs[b], dst, wsems[b]).wait()


def kernel(input_seqs, table):
    idx = input_seqs.reshape(NW, NCHUNK, CHUNK).astype(jnp.int32)
    out = _embed_gather(idx, table)
    return out.reshape(BATCH, HIST, EMBED)


# speedup vs baseline: 3.3509x; 1.0050x over previous
import functools

import jax
import jax.numpy as jnp
from jax import lax
from jax.experimental import pallas as pl
from jax.experimental.pallas import tpu as pltpu
from jax.experimental.pallas import tpu_sc as plsc

VOCAB = 100000
EMBED = 128
BATCH = 4096
HIST = 50

NTOT = BATCH * HIST
NW = 32
PER_W = NTOT // NW
CHUNK = 128
NCHUNK = PER_W // CHUNK


@functools.partial(
    pl.kernel,
    mesh=plsc.VectorSubcoreMesh(core_axis_name="c", subcore_axis_name="s"),
    out_type=jax.ShapeDtypeStruct((NTOT, EMBED), jnp.float32),
    scratch_types=[
        pltpu.VMEM((NCHUNK, CHUNK), jnp.int32),
        pltpu.VMEM((CHUNK, EMBED), jnp.float32),
        pltpu.VMEM((CHUNK, EMBED), jnp.float32),
        pltpu.VMEM((CHUNK, EMBED), jnp.float32),
        pltpu.VMEM((CHUNK, EMBED), jnp.float32),
        pltpu.SemaphoreType.DMA,
        pltpu.SemaphoreType.DMA,
        pltpu.SemaphoreType.DMA,
        pltpu.SemaphoreType.DMA,
        pltpu.SemaphoreType.DMA,
        pltpu.SemaphoreType.DMA,
        pltpu.SemaphoreType.DMA,
        pltpu.SemaphoreType.DMA,
    ],
)
def _embed_gather(idx_hbm, table_hbm, out_hbm, idx_v, r0, r1, r2, r3,
                  g0, g1, g2, g3, w0, w1, w2, w3):
    wid = lax.axis_index("s") * 2 + lax.axis_index("c")
    base = wid * PER_W
    bufs = (r0, r1, r2, r3)
    gsems = (g0, g1, g2, g3)
    wsems = (w0, w1, w2, w3)
    pltpu.sync_copy(idx_hbm.at[wid], idx_v)

    def wb_dst(c):
        return out_hbm.at[pl.ds(base + c * CHUNK, CHUNK)]

    pltpu.async_copy(table_hbm.at[idx_v.at[0]], r0, g0)
    pltpu.async_copy(table_hbm.at[idx_v.at[1]], r1, g1)

    def body(i, carry):
        for b in range(4):
            s = i * 4 + b
            buf, gs, ws = bufs[b], gsems[b], wsems[b]
            nbuf = bufs[(b + 2) % 4]
            nws = wsems[(b + 2) % 4]
            pltpu.make_async_copy(table_hbm.at[idx_v.at[s]], buf, gs).wait()
            pltpu.async_copy(buf, wb_dst(s), ws)

            @pl.when(s >= 2)
            def _():
                pltpu.make_async_copy(nbuf, wb_dst(s - 2), nws).wait()

            pltpu.async_copy(table_hbm.at[idx_v.at[s + 2]], nbuf,
                             gsems[(b + 2) % 4])
        return carry

    lax.fori_loop(0, (NCHUNK - 2) // 4, body, 0)

    for s in (NCHUNK - 2, NCHUNK - 1):
        b = s % 4
        pltpu.make_async_copy(table_hbm.at[idx_v.at[s]], bufs[b],
                              gsems[b]).wait()
        pltpu.async_copy(bufs[b], wb_dst(s), wsems[b])
        pltpu.make_async_copy(bufs[(b + 2) % 4], wb_dst(s - 2),
                              wsems[(b + 2) % 4]).wait()

    for s in (NCHUNK - 2, NCHUNK - 1):
        b = s % 4
        pltpu.make_async_copy(bufs[b], wb_dst(s), wsems[b]).wait()


def kernel(input_seqs, table):
    idx = input_seqs.reshape(NW, NCHUNK, CHUNK).astype(jnp.int32)
    out = _embed_gather(idx, table)
    return out.reshape(BATCH, HIST, EMBED)
